# baseline (speedup 1.0000x reference)
import functools

import jax
import jax.numpy as jnp
from jax import lax
from jax.experimental import pallas as pl
from jax.experimental.pallas import tpu as pltpu
from jax.experimental.pallas import tpu_sc as plsc


def _stats_body(x_ref, sum_ref, sq_ref):
    i = pl.program_id(0)

    @pl.when(i == 0)
    def _():
        sum_ref[...] = jnp.zeros_like(sum_ref)
        sq_ref[...] = jnp.zeros_like(sq_ref)

    x = x_ref[...]
    s = jnp.sum(x, axis=0, keepdims=True)
    q = jnp.sum(x * x, axis=0, keepdims=True)
    sum_ref[...] += jnp.broadcast_to(s, sum_ref.shape)
    sq_ref[...] += jnp.broadcast_to(q, sq_ref.shape)


def _col_stats(x, chunk):
    m, d = x.shape
    out = pl.pallas_call(
        _stats_body,
        grid=(m // chunk,),
        in_specs=[pl.BlockSpec((chunk, d), lambda i: (i, 0))],
        out_specs=[pl.BlockSpec((8, d), lambda i: (0, 0)),
                   pl.BlockSpec((8, d), lambda i: (0, 0))],
        out_shape=[jax.ShapeDtypeStruct((8, d), jnp.float32),
                   jax.ShapeDtypeStruct((8, d), jnp.float32)],
    )(x)
    return out[0][0], out[1][0]


def _node_body(h_ref, msc_ref, gnw_ref, gnb_ref, w4_ref, b4_ref,
               ah_ref, tbc_ref, td_ref):
    h = h_ref[...]
    n = h.shape[0]
    npad = td_ref.shape[0]
    pad = jnp.zeros((npad - n, 128), jnp.float32)
    mean = jnp.mean(h, axis=0, keepdims=True)
    sub = h - mean * msc_ref[...]
    std = jnp.sqrt(jnp.mean(sub * sub, axis=0, keepdims=True) + 1e-6)
    hn = gnw_ref[...] * sub / std + gnb_ref[...]
    ah_ref[...] = (jnp.dot(hn, w4_ref[0], preferred_element_type=jnp.float32)
                   + b4_ref[0, 0:1, :])
    vb = (jnp.dot(hn, w4_ref[1], preferred_element_type=jnp.float32)
          + b4_ref[1, 0:1, :])
    vc = (jnp.dot(hn, w4_ref[2], preferred_element_type=jnp.float32)
          + b4_ref[2, 0:1, :])
    vd = (jnp.dot(hn, w4_ref[3], preferred_element_type=jnp.float32)
          + b4_ref[3, 0:1, :])
    tbc_ref[...] = jnp.concatenate(
        [jnp.concatenate([vc[:, :64], vb[:, :64]], axis=1), pad,
         jnp.concatenate([vc[:, 64:], vb[:, 64:]], axis=1), pad], axis=0)
    td_ref[...] = jnp.concatenate([vd, pad], axis=0)


def _node_stage(h, msc, gnw, gnb, w4, b4, npad):
    n, d = h.shape
    outs = pl.pallas_call(
        _node_body,
        grid=(1,),
        in_specs=[pl.BlockSpec(h.shape, lambda i: (0, 0)),
                  pl.BlockSpec((1, d), lambda i: (0, 0)),
                  pl.BlockSpec((1, d), lambda i: (0, 0)),
                  pl.BlockSpec((1, d), lambda i: (0, 0)),
                  pl.BlockSpec(w4.shape, lambda i: (0, 0, 0)),
                  pl.BlockSpec(b4.shape, lambda i: (0, 0, 0))],
        out_specs=[pl.BlockSpec((n, d), lambda i: (0, 0)),
                   pl.BlockSpec((2 * npad, d), lambda i: (0, 0)),
                   pl.BlockSpec((npad, d), lambda i: (0, 0))],
        out_shape=[jax.ShapeDtypeStruct((n, d), jnp.float32),
                   jax.ShapeDtypeStruct((2 * npad, d), jnp.float32),
                   jax.ShapeDtypeStruct((npad, d), jnp.float32)],
    )(h, msc, gnw, gnb, w4, b4)
    return outs


def _pack_body(x_ref, a_ref, c_ref, o_ref):
    x = x_ref[...]
    mm = (jnp.dot(x, a_ref[...],
                  preferred_element_type=jnp.float32) + c_ref[0:1, :])
    o_ref[...] = jnp.concatenate(
        [mm[:, :64], x[:, :64], mm[:, 64:], x[:, 64:]], axis=1)


def _edge_pack(e, a_fold, c8, chunk):
    m, d = e.shape
    nb = m // chunk
    return pl.pallas_call(
        _pack_body,
        grid=(nb,),
        in_specs=[pl.BlockSpec((chunk, d), lambda i: (i, 0)),
                  pl.BlockSpec((d, d), lambda i: (0, 0)),
                  pl.BlockSpec((8, d), lambda i: (0, 0))],
        out_specs=pl.BlockSpec((chunk, 2 * d), lambda i: (i, 0)),
        out_shape=jax.ShapeDtypeStruct((m, 2 * d), jnp.float32),
    )(e, a_fold, c8)


def _node2_body(ah_ref, v0_ref, v1_ref, h_ref, p_ref, g_ref, ho_ref):
    g = g_ref[0, 0]
    n = h_ref.shape[0]
    sh = jnp.concatenate([v0_ref[:n, 0:64], v1_ref[:n, 0:64]], axis=1)
    ss = jnp.concatenate([v0_ref[:n, 64:128], v1_ref[:n, 64:128]], axis=1)
    h_new = ah_ref[...] + sh / (ss + 1e-10)
    h1 = h_ref[...] + h_new * g
    msc = p_ref[0, 0:1, :]
    gnw = p_ref[0, 1:2, :]
    gnb = p_ref[0, 2:3, :]
    b1 = p_ref[0, 3:4, :]
    b2 = p_ref[0, 4:5, :]
    w1 = p_ref[1]
    w2 = p_ref[2]
    mean = jnp.mean(h1, axis=0, keepdims=True)
    sub = h1 - mean * msc
    std = jnp.sqrt(jnp.mean(sub * sub, axis=0, keepdims=True) + 1e-6)
    hn2 = gnw * sub / std + gnb
    hf = jnp.maximum(jnp.dot(hn2, w1, preferred_element_type=jnp.float32)
                     + b1, 0.0)
    hf = jnp.dot(hf, w2, preferred_element_type=jnp.float32) + b2
    ho_ref[...] = h1 + hf * g


def _node2_stage(ah, ns, h, pk, g, npad):
    n, d = h.shape
    return pl.pallas_call(
        _node2_body,
        grid=(1,),
        in_specs=[pl.BlockSpec((n, d), lambda i: (0, 0)),
                  pl.BlockSpec((npad, d), lambda i: (0, 0)),
                  pl.BlockSpec((npad, d), lambda i: (1, 0)),
                  pl.BlockSpec((n, d), lambda i: (0, 0)),
                  pl.BlockSpec(pk.shape, lambda i: (0, 0, 0)),
                  pl.BlockSpec((1, 1), lambda i: (0, 0),
                               memory_space=pltpu.SMEM)],
        out_specs=pl.BlockSpec((n, d), lambda i: (0, 0)),
        out_shape=jax.ShapeDtypeStruct((n, d), jnp.float32),
    )(ah, ns, ns, h, pk, g)


def _edge2_body(l_ref, r_ref, aff_ref, w1_ref, w2_ref, b_ref, g_ref, eo_ref):
    g = g_ref[0, 0]
    e1 = jnp.concatenate([l_ref[...], r_ref[...]], axis=1)
    en2 = e1 * aff_ref[0:1, :] + aff_ref[4:5, :]
    ef = jnp.maximum(jnp.dot(en2, w1_ref[...],
                             preferred_element_type=jnp.float32) + b_ref[0:1, :],
                     0.0)
    ef = jnp.dot(ef, w2_ref[...], preferred_element_type=jnp.float32) + b_ref[4:5, :]
    eo_ref[...] = e1 + ef * g


def _edge2_stage(e1s, aff, w1, w2, b, g, m, chunk):
    d = 128
    nb = m // chunk
    return pl.pallas_call(
        _edge2_body,
        grid=(nb,),
        in_specs=[pl.BlockSpec((chunk, d // 2), lambda i: (i, 0)),
                  pl.BlockSpec((chunk, d // 2), lambda i: (nb + i, 0)),
                  pl.BlockSpec((8, d), lambda i: (0, 0)),
                  pl.BlockSpec((d, d), lambda i: (0, 0)),
                  pl.BlockSpec((d, d), lambda i: (0, 0)),
                  pl.BlockSpec((8, d), lambda i: (0, 0)),
                  pl.BlockSpec((1, 1), lambda i: (0, 0),
                               memory_space=pltpu.SMEM)],
        out_specs=pl.BlockSpec((chunk, d), lambda i: (i, 0)),
        out_shape=jax.ShapeDtypeStruct((m, d), jnp.float32),
    )(e1s, e1s, aff, w1, w2, b, g)


_C = 80


def _sc_middle(tbc, td, pack, src_i, dst_i, zrs, gvec, npad, m):
    epw = m // 16
    chunks = epw // _C
    mesh = plsc.VectorSubcoreMesh(core_axis_name="c", subcore_axis_name="s")
    stripe = npad // 16

    @functools.partial(
        pl.kernel, mesh=mesh,
        out_type=[jax.ShapeDtypeStruct((2 * m, 64), jnp.float32),
                  jax.ShapeDtypeStruct((2 * npad, 128), jnp.float32),
                  jax.ShapeDtypeStruct((256, 128), jnp.float32)],
        scratch_types=[pltpu.VMEM((_C,), jnp.int32),
                       pltpu.VMEM((_C,), jnp.int32),
                       pltpu.VMEM((_C, 128), jnp.float32),
                       pltpu.VMEM((_C, 128), jnp.float32),
                       pltpu.VMEM((_C, 128), jnp.float32),
                       pltpu.VMEM((_C, 64), jnp.float32),
                       pltpu.VMEM((16,), jnp.float32),
                       pltpu.VMEM((128,), jnp.float32),
                       pltpu.VMEM_SHARED((npad, 128), jnp.float32),
                       pltpu.SemaphoreType.DMA,
                       pltpu.SemaphoreType.DMA],
    )
    def body(tbc_h, td_h, pk_h, src_h, dst_h, z_h, g_h,
             e1_h, ns_h, st_h,
             six, dix, bG, bD, bP, bO, gv, stv, acc, sem, sem2):
        c = lax.axis_index("c")
        s = lax.axis_index("s")
        cn = c * npad
        pltpu.sync_copy(z_h.at[pl.ds(s * stripe, stripe)],
                        acc.at[pl.ds(s * stripe, stripe)])
        pltpu.sync_copy(g_h, gv)
        plsc.subcore_barrier()
        g16 = gv[...]
        zero = jnp.zeros((16,), jnp.float32)
        base0 = s * epw
        c_is0 = c == 0

        def chunk_fn(k, carry):
            be = base0 + k * _C
            pltpu.sync_copy(src_h.at[pl.ds(be, _C)], six)
            pltpu.sync_copy(dst_h.at[pl.ds(be, _C)], dix)

            def adj(j, _):
                sl = pl.ds(j * 16, 16)
                six[sl] = six[sl] + cn
                return 0
            lax.fori_loop(0, _C // 16, adj, 0, unroll=True)

            hc = _C // 2
            g1a = pltpu.async_copy(tbc_h.at[six.at[pl.ds(0, hc)]],
                                   bG.at[pl.ds(0, hc)], sem)
            g1b = pltpu.async_copy(td_h.at[dix.at[pl.ds(0, hc)]],
                                   bD.at[pl.ds(0, hc)], sem)
            g1c = pltpu.async_copy(
                pk_h.at[pl.ds(be, hc), pl.ds(c * 128, 128)],
                bP.at[pl.ds(0, hc)], sem)
            g2a = pltpu.async_copy(tbc_h.at[six.at[pl.ds(hc, hc)]],
                                   bG.at[pl.ds(hc, hc)], sem2)
            g2b = pltpu.async_copy(td_h.at[dix.at[pl.ds(hc, hc)]],
                                   bD.at[pl.ds(hc, hc)], sem2)
            g2c = pltpu.async_copy(
                pk_h.at[pl.ds(be + hc, hc), pl.ds(c * 128, 128)],
                bP.at[pl.ds(hc, hc)], sem2)

            def row(r, st):
                st = list(st)
                for qi in range(4):
                    o = qi * 16
                    dlo = bD[r, pl.ds(o, 16)]
                    dhi = bD[r, pl.ds(64 + o, 16)]
                    dhv = jnp.where(c_is0, dlo, dhi)
                    enw = bG[r, pl.ds(o, 16)] + dhv + bP[r, pl.ds(o, 16)]
                    e1v = bP[r, pl.ds(64 + o, 16)] + enw * g16
                    bO[r, pl.ds(o, 16)] = e1v
                    ax = jnp.minimum(jnp.abs(enw), 30.0)
                    den = 1.0 + jnp.exp(-ax)
                    r0 = 1.411764705882 - 0.470588235294 * den
                    r0 = r0 * (2.0 - den * r0)
                    r0 = r0 * (2.0 - den * r0)
                    r0 = r0 * (2.0 - den * r0)
                    sg = jnp.where(enw >= 0.0, r0, 1.0 - r0)
                    bhv = bG[r, pl.ds(64 + o, 16)]
                    bG[r, pl.ds(64 + o, 16)] = sg
                    bG[r, pl.ds(o, 16)] = bhv * sg
                    st[qi] = st[qi] + e1v
                    st[4 + qi] = st[4 + qi] + e1v * e1v
                return tuple(st)

            g1a.wait()
            g1b.wait()
            g1c.wait()
            carry = lax.fori_loop(0, hc, row, carry, unroll=2)
            pltpu.sync_copy(bO.at[pl.ds(0, hc)],
                            e1_h.at[pl.ds(c * m + be, hc)])
            g2a.wait()
            g2b.wait()
            g2c.wait()
            carry = lax.fori_loop(hc, _C, row, carry, unroll=2)
            pltpu.sync_copy(bO.at[pl.ds(hc, hc)],
                            e1_h.at[pl.ds(c * m + be + hc, hc)])
            pltpu.sync_copy(bG, acc.at[dix], add=True)
            return carry

        fin = lax.fori_loop(0, chunks, chunk_fn, (zero,) * 8)
        for qi in range(4):
            stv[pl.ds(qi * 16, 16)] = fin[qi]
            stv[pl.ds(64 + qi * 16, 16)] = fin[4 + qi]
        wid = (c * 16 + s) * 8
        pltpu.sync_copy(stv, st_h.at[wid])
        plsc.subcore_barrier()
        pltpu.sync_copy(acc.at[pl.ds(s * stripe, stripe)],
                        ns_h.at[pl.ds(cn + s * stripe, stripe)])

    return body(tbc, td, pack, src_i, dst_i, zrs, gvec)


def kernel(h, e, edge_index, params):
    n, d = h.shape
    m = e.shape[0]
    src = edge_index[0]
    dst = edge_index[1]
    g = params['rz_g'][0]
    gs = jnp.reshape(g, (1, 1))

    es, eq = _col_stats(e, 8000)
    emean = es / m
    evar = eq / m - 2.0 * params['n1e_m'] * emean * emean + (
        params['n1e_m'] * emean) ** 2
    estd = jnp.sqrt(evar + 1e-6)
    scale = params['n1e_w'] / estd
    shift = params['n1e_b'] - scale * emean * params['n1e_m']
    elw = params['El_w'].T
    a_fold = scale[:, None] * elw
    c_fold = params['El_b'] + shift @ elw
    c8 = jnp.broadcast_to(c_fold[None, :], (8, d))

    w4 = jnp.stack([params['A_w'].T, params['B_w'].T,
                    params['C_w'].T, params['Dl_w'].T])
    b4 = jnp.stack([jnp.broadcast_to(params[k + '_b'][None, :], (8, d))
                    for k in ('A', 'B', 'C', 'Dl')])
    npad = 10240
    ah, tbc, td = _node_stage(
        h, params['n1h_m'][None, :], params['n1h_w'][None, :],
        params['n1h_b'][None, :], w4, b4, npad)

    pack = _edge_pack(e, a_fold, c8, 8000)

    zrs = jnp.zeros((npad, 128), jnp.float32)
    gvec = jnp.broadcast_to(params['rz_g'], (16,))
    e1s, ns, stats = _sc_middle(tbc, td, pack, src, dst, zrs, gvec, npad, m)
    stats = stats[::8]

    pk = jnp.stack([
        jnp.concatenate([params['n2h_m'][None, :], params['n2h_w'][None, :],
                         params['n2h_b'][None, :], params['fh1_b'][None, :],
                         params['fh2_b'][None, :],
                         jnp.zeros((d - 5, d), jnp.float32)], axis=0),
        params['fh1_w'].T, params['fh2_w'].T])
    h_out = _node2_stage(ah, ns, h, pk, gs, npad)

    s1 = jnp.concatenate([jnp.sum(stats[0:16, 0:64], axis=0),
                          jnp.sum(stats[16:32, 0:64], axis=0)])
    q1 = jnp.concatenate([jnp.sum(stats[0:16, 64:128], axis=0),
                          jnp.sum(stats[16:32, 64:128], axis=0)])
    m2 = s1 / m
    v2 = q1 / m - 2.0 * params['n2e_m'] * m2 * m2 + (params['n2e_m'] * m2) ** 2
    std2 = jnp.sqrt(v2 + 1e-6)
    sc2 = params['n2e_w'] / std2
    sh2 = params['n2e_b'] - sc2 * m2 * params['n2e_m']
    aff = jnp.concatenate([jnp.broadcast_to(sc2[None, :], (4, d)),
                           jnp.broadcast_to(sh2[None, :], (4, d))], axis=0)
    b2pack = jnp.concatenate(
        [jnp.broadcast_to(params['fe1_b'][None, :], (4, d)),
         jnp.broadcast_to(params['fe2_b'][None, :], (4, d))], axis=0)
    e_out = _edge2_stage(e1s, aff, params['fe1_w'].T, params['fe2_w'].T,
                         b2pack, gs, m, 8000)
    return (h_out, e_out)


# speedup vs baseline: 2.4697x; 2.4697x over previous
import functools

import jax
import jax.numpy as jnp
from jax import lax
from jax.experimental import pallas as pl
from jax.experimental.pallas import tpu as pltpu
from jax.experimental.pallas import tpu_sc as plsc


def _stats_body(x_ref, sum_ref, sq_ref):
    i = pl.program_id(0)

    @pl.when(i == 0)
    def _():
        sum_ref[...] = jnp.zeros_like(sum_ref)
        sq_ref[...] = jnp.zeros_like(sq_ref)

    x = x_ref[...]
    s = jnp.sum(x, axis=0, keepdims=True)
    q = jnp.sum(x * x, axis=0, keepdims=True)
    sum_ref[...] += jnp.broadcast_to(s, sum_ref.shape)
    sq_ref[...] += jnp.broadcast_to(q, sq_ref.shape)


def _col_stats(x, chunk):
    m, d = x.shape
    out = pl.pallas_call(
        _stats_body,
        grid=(m // chunk,),
        in_specs=[pl.BlockSpec((chunk, d), lambda i: (i, 0))],
        out_specs=[pl.BlockSpec((8, d), lambda i: (0, 0)),
                   pl.BlockSpec((8, d), lambda i: (0, 0))],
        out_shape=[jax.ShapeDtypeStruct((8, d), jnp.float32),
                   jax.ShapeDtypeStruct((8, d), jnp.float32)],
    )(x)
    return out[0][0], out[1][0]


def _stats2_body(x_ref, sl_ref, ql_ref, sr_ref, qr_ref, *, nb):
    i = pl.program_id(0)

    @pl.when(i == 0)
    def _():
        sl_ref[...] = jnp.zeros_like(sl_ref)
        ql_ref[...] = jnp.zeros_like(ql_ref)
        sr_ref[...] = jnp.zeros_like(sr_ref)
        qr_ref[...] = jnp.zeros_like(qr_ref)

    x = x_ref[...]
    s = jnp.broadcast_to(jnp.sum(x, axis=0, keepdims=True), sl_ref.shape)
    q = jnp.broadcast_to(jnp.sum(x * x, axis=0, keepdims=True), ql_ref.shape)

    @pl.when(i < nb)
    def _():
        sl_ref[...] += s
        ql_ref[...] += q

    @pl.when(i >= nb)
    def _():
        sr_ref[...] += s
        qr_ref[...] += q


def _col_stats_split(x2, chunk):
    m2, dh = x2.shape
    nb = (m2 // 2) // chunk
    out = pl.pallas_call(
        functools.partial(_stats2_body, nb=nb),
        grid=(2 * nb,),
        in_specs=[pl.BlockSpec((chunk, dh), lambda i: (i, 0))],
        out_specs=[pl.BlockSpec((8, dh), lambda i: (0, 0))] * 4,
        out_shape=[jax.ShapeDtypeStruct((8, dh), jnp.float32)] * 4,
    )(x2)
    s1 = jnp.concatenate([out[0][0], out[2][0]])
    q1 = jnp.concatenate([out[1][0], out[3][0]])
    return s1, q1


def _node_body(h_ref, msc_ref, gnw_ref, gnb_ref, w4_ref, b4_ref,
               ah_ref, tbc_ref, td_ref):
    h = h_ref[...]
    n = h.shape[0]
    npad = td_ref.shape[0] // 2
    pad = jnp.zeros((npad - n, 128), jnp.float32)
    mean = jnp.mean(h, axis=0, keepdims=True)
    sub = h - mean * msc_ref[...]
    std = jnp.sqrt(jnp.mean(sub * sub, axis=0, keepdims=True) + 1e-6)
    hn = gnw_ref[...] * sub / std + gnb_ref[...]
    ah_ref[...] = (jnp.dot(hn, w4_ref[0], preferred_element_type=jnp.float32)
                   + b4_ref[0, 0:1, :])
    vb = (jnp.dot(hn, w4_ref[1], preferred_element_type=jnp.float32)
          + b4_ref[1, 0:1, :])
    vc = (jnp.dot(hn, w4_ref[2], preferred_element_type=jnp.float32)
          + b4_ref[2, 0:1, :])
    vd = (jnp.dot(hn, w4_ref[3], preferred_element_type=jnp.float32)
          + b4_ref[3, 0:1, :])
    tbc_ref[...] = jnp.concatenate(
        [jnp.concatenate([vc[:, :64], vb[:, :64]], axis=1), pad,
         jnp.concatenate([vc[:, 64:], vb[:, 64:]], axis=1), pad], axis=0)
    td_ref[...] = jnp.concatenate(
        [jnp.concatenate([vd[:, :64], vd[:, :64]], axis=1), pad,
         jnp.concatenate([vd[:, 64:], vd[:, 64:]], axis=1), pad], axis=0)


def _node_stage(h, msc, gnw, gnb, w4, b4, npad):
    n, d = h.shape
    outs = pl.pallas_call(
        _node_body,
        grid=(1,),
        in_specs=[pl.BlockSpec(h.shape, lambda i: (0, 0)),
                  pl.BlockSpec((1, d), lambda i: (0, 0)),
                  pl.BlockSpec((1, d), lambda i: (0, 0)),
                  pl.BlockSpec((1, d), lambda i: (0, 0)),
                  pl.BlockSpec(w4.shape, lambda i: (0, 0, 0)),
                  pl.BlockSpec(b4.shape, lambda i: (0, 0, 0))],
        out_specs=[pl.BlockSpec((n, d), lambda i: (0, 0)),
                   pl.BlockSpec((2 * npad, d), lambda i: (0, 0)),
                   pl.BlockSpec((2 * npad, d), lambda i: (0, 0))],
        out_shape=[jax.ShapeDtypeStruct((n, d), jnp.float32),
                   jax.ShapeDtypeStruct((2 * npad, d), jnp.float32),
                   jax.ShapeDtypeStruct((2 * npad, d), jnp.float32)],
    )(h, msc, gnw, gnb, w4, b4)
    return outs


def _pack_body(x_ref, a_ref, c_ref, o_ref):
    x = x_ref[...]
    mm = (jnp.dot(x, a_ref[...],
                  preferred_element_type=jnp.float32) + c_ref[0:1, :])
    o_ref[...] = jnp.concatenate(
        [mm[:, :64], x[:, :64], mm[:, 64:], x[:, 64:]], axis=1)


def _edge_pack(e, a_fold, c8, chunk):
    m, d = e.shape
    nb = m // chunk
    return pl.pallas_call(
        _pack_body,
        grid=(nb,),
        in_specs=[pl.BlockSpec((chunk, d), lambda i: (i, 0)),
                  pl.BlockSpec((d, d), lambda i: (0, 0)),
                  pl.BlockSpec((8, d), lambda i: (0, 0))],
        out_specs=pl.BlockSpec((chunk, 2 * d), lambda i: (i, 0)),
        out_shape=jax.ShapeDtypeStruct((m, 2 * d), jnp.float32),
    )(e, a_fold, c8)


def _node2_body(ah_ref, v0_ref, v1_ref, h_ref, p_ref, g_ref, ho_ref):
    g = g_ref[0, 0]
    n = h_ref.shape[0]
    sh = jnp.concatenate([v0_ref[:n, 0:64], v1_ref[:n, 0:64]], axis=1)
    ss = jnp.concatenate([v0_ref[:n, 64:128], v1_ref[:n, 64:128]], axis=1)
    h_new = ah_ref[...] + sh / (ss + 1e-10)
    h1 = h_ref[...] + h_new * g
    msc = p_ref[0, 0:1, :]
    gnw = p_ref[0, 1:2, :]
    gnb = p_ref[0, 2:3, :]
    b1 = p_ref[0, 3:4, :]
    b2 = p_ref[0, 4:5, :]
    w1 = p_ref[1]
    w2 = p_ref[2]
    mean = jnp.mean(h1, axis=0, keepdims=True)
    sub = h1 - mean * msc
    std = jnp.sqrt(jnp.mean(sub * sub, axis=0, keepdims=True) + 1e-6)
    hn2 = gnw * sub / std + gnb
    hf = jnp.maximum(jnp.dot(hn2, w1, preferred_element_type=jnp.float32)
                     + b1, 0.0)
    hf = jnp.dot(hf, w2, preferred_element_type=jnp.float32) + b2
    ho_ref[...] = h1 + hf * g


def _node2_stage(ah, ns, h, pk, g, npad):
    n, d = h.shape
    return pl.pallas_call(
        _node2_body,
        grid=(1,),
        in_specs=[pl.BlockSpec((n, d), lambda i: (0, 0)),
                  pl.BlockSpec((npad, d), lambda i: (0, 0)),
                  pl.BlockSpec((npad, d), lambda i: (1, 0)),
                  pl.BlockSpec((n, d), lambda i: (0, 0)),
                  pl.BlockSpec(pk.shape, lambda i: (0, 0, 0)),
                  pl.BlockSpec((1, 1), lambda i: (0, 0),
                               memory_space=pltpu.SMEM)],
        out_specs=pl.BlockSpec((n, d), lambda i: (0, 0)),
        out_shape=jax.ShapeDtypeStruct((n, d), jnp.float32),
    )(ah, ns, ns, h, pk, g)


def _edge2_body(l_ref, r_ref, aff_ref, w1_ref, w2_ref, b_ref, g_ref, eo_ref):
    g = g_ref[0, 0]
    e1 = jnp.concatenate([l_ref[...], r_ref[...]], axis=1)
    en2 = e1 * aff_ref[0:1, :] + aff_ref[4:5, :]
    ef = jnp.maximum(jnp.dot(en2, w1_ref[...],
                             preferred_element_type=jnp.float32) + b_ref[0:1, :],
                     0.0)
    ef = jnp.dot(ef, w2_ref[...], preferred_element_type=jnp.float32) + b_ref[4:5, :]
    eo_ref[...] = e1 + ef * g


def _edge2_stage(e1s, aff, w1, w2, b, g, m, chunk):
    d = 128
    nb = m // chunk
    return pl.pallas_call(
        _edge2_body,
        grid=(nb,),
        in_specs=[pl.BlockSpec((chunk, d // 2), lambda i: (i, 0)),
                  pl.BlockSpec((chunk, d // 2), lambda i: (nb + i, 0)),
                  pl.BlockSpec((8, d), lambda i: (0, 0)),
                  pl.BlockSpec((d, d), lambda i: (0, 0)),
                  pl.BlockSpec((d, d), lambda i: (0, 0)),
                  pl.BlockSpec((8, d), lambda i: (0, 0)),
                  pl.BlockSpec((1, 1), lambda i: (0, 0),
                               memory_space=pltpu.SMEM)],
        out_specs=pl.BlockSpec((chunk, d), lambda i: (i, 0)),
        out_shape=jax.ShapeDtypeStruct((m, d), jnp.float32),
    )(e1s, e1s, aff, w1, w2, b, g)


_C = 80


def _sc_middle(tbc, td, pack, src_i, dst_i, zrs, gvec, npad, m):
    epw = m // 16
    chunks = epw // _C
    mesh = plsc.VectorSubcoreMesh(core_axis_name="c", subcore_axis_name="s")
    stripe = npad // 16

    @functools.partial(
        pl.kernel, mesh=mesh,
        out_type=[jax.ShapeDtypeStruct((2 * m, 64), jnp.float32),
                  jax.ShapeDtypeStruct((2 * npad, 128), jnp.float32)],
        scratch_types=[pltpu.VMEM((_C,), jnp.int32),
                       pltpu.VMEM((_C,), jnp.int32),
                       pltpu.VMEM((_C,), jnp.int32),
                       pltpu.VMEM((_C, 128), jnp.float32),
                       pltpu.VMEM((_C, 128), jnp.float32),
                       pltpu.VMEM((_C, 128), jnp.float32),
                       pltpu.VMEM((_C, 64), jnp.float32),
                       pltpu.VMEM((16,), jnp.float32),
                       pltpu.VMEM_SHARED((npad, 128), jnp.float32),
                       pltpu.SemaphoreType.DMA,
                       pltpu.SemaphoreType.DMA],
    )
    def body(tbc_h, td_h, pk_h, src_h, dst_h, z_h, g_h,
             e1_h, ns_h,
             six, dix, djx, bG, bD, bP, bO, gv, acc, sem, sem2):
        c = lax.axis_index("c")
        s = lax.axis_index("s")
        cn = c * npad
        pltpu.sync_copy(z_h.at[pl.ds(s * stripe, stripe)],
                        acc.at[pl.ds(s * stripe, stripe)])
        pltpu.sync_copy(g_h, gv)
        plsc.subcore_barrier()
        g16 = gv[...]
        base0 = s * epw

        def chunk_fn(k, _):
            be = base0 + k * _C
            pltpu.sync_copy(src_h.at[pl.ds(be, _C)], six)
            pltpu.sync_copy(dst_h.at[pl.ds(be, _C)], dix)

            def adj(j, _):
                sl = pl.ds(j * 16, 16)
                six[sl] = six[sl] + cn
                djx[sl] = dix[sl] + cn
                return 0
            lax.fori_loop(0, _C // 16, adj, 0, unroll=True)

            hc = _C // 2
            g1a = pltpu.async_copy(tbc_h.at[six.at[pl.ds(0, hc)]],
                                   bG.at[pl.ds(0, hc)], sem)
            g1b = pltpu.async_copy(td_h.at[djx.at[pl.ds(0, hc)]],
                                   bD.at[pl.ds(0, hc)], sem)
            g1c = pltpu.async_copy(
                pk_h.at[pl.ds(be, hc), pl.ds(c * 128, 128)],
                bP.at[pl.ds(0, hc)], sem)
            g2a = pltpu.async_copy(tbc_h.at[six.at[pl.ds(hc, hc)]],
                                   bG.at[pl.ds(hc, hc)], sem2)
            g2b = pltpu.async_copy(td_h.at[djx.at[pl.ds(hc, hc)]],
                                   bD.at[pl.ds(hc, hc)], sem2)
            g2c = pltpu.async_copy(
                pk_h.at[pl.ds(be + hc, hc), pl.ds(c * 128, 128)],
                bP.at[pl.ds(hc, hc)], sem2)

            def row_half(lo):
                @functools.partial(plsc.parallel_loop, lo, lo + hc,
                                   unroll=2)
                def _(r):
                    for qi in range(4):
                        o = qi * 16
                        enw = (bG[r, pl.ds(o, 16)] + bD[r, pl.ds(o, 16)]
                               + bP[r, pl.ds(o, 16)])
                        e1v = bP[r, pl.ds(64 + o, 16)] + enw * g16
                        bO[r, pl.ds(o, 16)] = e1v
                        sg = 1.0 / (1.0 + jnp.exp(-enw))
                        bhv = bG[r, pl.ds(64 + o, 16)]
                        bG[r, pl.ds(64 + o, 16)] = sg
                        bG[r, pl.ds(o, 16)] = bhv * sg

            g1a.wait()
            g1b.wait()
            g1c.wait()
            row_half(0)
            pltpu.sync_copy(bO.at[pl.ds(0, hc)],
                            e1_h.at[pl.ds(c * m + be, hc)])
            g2a.wait()
            g2b.wait()
            g2c.wait()
            row_half(hc)
            pltpu.sync_copy(bO.at[pl.ds(hc, hc)],
                            e1_h.at[pl.ds(c * m + be + hc, hc)])
            pltpu.sync_copy(bG, acc.at[dix], add=True)
            return 0

        lax.fori_loop(0, chunks, chunk_fn, 0)
        plsc.subcore_barrier()
        pltpu.sync_copy(acc.at[pl.ds(s * stripe, stripe)],
                        ns_h.at[pl.ds(cn + s * stripe, stripe)])

    return body(tbc, td, pack, src_i, dst_i, zrs, gvec)


def kernel(h, e, edge_index, params):
    n, d = h.shape
    m = e.shape[0]
    src = edge_index[0]
    dst = edge_index[1]
    g = params['rz_g'][0]
    gs = jnp.reshape(g, (1, 1))

    es, eq = _col_stats(e, 8000)
    emean = es / m
    evar = eq / m - 2.0 * params['n1e_m'] * emean * emean + (
        params['n1e_m'] * emean) ** 2
    estd = jnp.sqrt(evar + 1e-6)
    scale = params['n1e_w'] / estd
    shift = params['n1e_b'] - scale * emean * params['n1e_m']
    elw = params['El_w'].T
    a_fold = scale[:, None] * elw
    c_fold = params['El_b'] + shift @ elw
    c8 = jnp.broadcast_to(c_fold[None, :], (8, d))

    w4 = jnp.stack([params['A_w'].T, params['B_w'].T,
                    params['C_w'].T, params['Dl_w'].T])
    b4 = jnp.stack([jnp.broadcast_to(params[k + '_b'][None, :], (8, d))
                    for k in ('A', 'B', 'C', 'Dl')])
    npad = 10240
    ah, tbc, td = _node_stage(
        h, params['n1h_m'][None, :], params['n1h_w'][None, :],
        params['n1h_b'][None, :], w4, b4, npad)

    pack = _edge_pack(e, a_fold, c8, 8000)

    zrs = jnp.zeros((npad, 128), jnp.float32)
    gvec = jnp.broadcast_to(params['rz_g'], (16,))
    e1s, ns = _sc_middle(tbc, td, pack, src, dst, zrs, gvec, npad, m)

    pk = jnp.stack([
        jnp.concatenate([params['n2h_m'][None, :], params['n2h_w'][None, :],
                         params['n2h_b'][None, :], params['fh1_b'][None, :],
                         params['fh2_b'][None, :],
                         jnp.zeros((d - 5, d), jnp.float32)], axis=0),
        params['fh1_w'].T, params['fh2_w'].T])
    h_out = _node2_stage(ah, ns, h, pk, gs, npad)

    s1, q1 = _col_stats_split(e1s, 8000)
    m2 = s1 / m
    v2 = q1 / m - 2.0 * params['n2e_m'] * m2 * m2 + (params['n2e_m'] * m2) ** 2
    std2 = jnp.sqrt(v2 + 1e-6)
    sc2 = params['n2e_w'] / std2
    sh2 = params['n2e_b'] - sc2 * m2 * params['n2e_m']
    aff = jnp.concatenate([jnp.broadcast_to(sc2[None, :], (4, d)),
                           jnp.broadcast_to(sh2[None, :], (4, d))], axis=0)
    b2pack = jnp.concatenate(
        [jnp.broadcast_to(params['fe1_b'][None, :], (4, d)),
         jnp.broadcast_to(params['fe2_b'][None, :], (4, d))], axis=0)
    e_out = _edge2_stage(e1s, aff, params['fe1_w'].T, params['fe2_w'].T,
                         b2pack, gs, m, 8000)
    return (h_out, e_out)
